# baseline (speedup 1.0000x reference)
import functools

import jax
import jax.numpy as jnp
from jax import lax
from jax.experimental import pallas as pl
from jax.experimental.pallas import tpu as pltpu
from jax.experimental.pallas import tpu_sc as plsc

H, W, C = 50, 50, 512
PH, PW = 7, 7
NPIX = PH * PW
NWORK = 16
SLOTS = 4
LANES = 16
CCHUNKS = C // LANES


@functools.partial(
    pl.kernel,
    out_type=jax.ShapeDtypeStruct((NPIX, C), jnp.float32),
    mesh=plsc.VectorSubcoreMesh(core_axis_name="c", subcore_axis_name="s",
                                num_cores=1),
    scratch_types=[
        pltpu.VMEM((LANES,), jnp.float32),
        pltpu.VMEM((LANES,), jnp.int32),
        pltpu.VMEM((LANES, C), jnp.float32),
        pltpu.VMEM((SLOTS, C), jnp.float32),
        pltpu.SemaphoreType.DMA,
    ],
)
def _roi_pool_sc(img_hbm, rois_hbm, out_hbm, roi_v, idx_v, rows_v, res_v,
                 sem):
    wid = lax.axis_index("s")
    lanes = lax.iota(jnp.int32, LANES)

    pltpu.sync_copy(rois_hbm.at[pl.ds(0, LANES)], roi_v)
    rvi = roi_v[...].astype(jnp.int32)
    x0c = rvi[0]
    y0c = rvi[1]
    in_w = rvi[2] - x0c + 1
    in_h = rvi[3] - y0c + 1
    d_w = jnp.full((LANES,), in_w.astype(jnp.float32)) / jnp.float32(PW)
    d_h = jnp.full((LANES,), in_h.astype(jnp.float32)) / jnp.float32(PH)

    s_l = lax.div(lanes, 4)
    q_l = lax.rem(lanes, 4)
    p_l = jnp.minimum(SLOTS * wid + s_l, NPIX - 1)
    i_l = lax.div(p_l, PW)
    j_l = lax.rem(p_l, PW)
    sy = i_l.astype(jnp.float32) * d_h
    sx = j_l.astype(jnp.float32) * d_w
    fy0 = sy.astype(jnp.int32)
    fx0 = sx.astype(jnp.int32)
    wy = sy - fy0.astype(jnp.float32)
    wx = sx - fx0.astype(jnp.float32)
    fy1 = jnp.minimum(fy0 + 1, in_h - 1)
    fx1 = jnp.minimum(fx0 + 1, in_w - 1)
    yy = y0c + jnp.where(q_l < 2, fy0, fy1)
    xx = x0c + jnp.where(lax.rem(q_l, 2) == 0, fx0, fx1)
    idx_v[...] = yy * W + xx

    wgt = jnp.where(q_l < 2, 1.0 - wy, wy) * jnp.where(
        lax.rem(q_l, 2) == 0, 1.0 - wx, wx)

    pltpu.async_copy(img_hbm.at[idx_v], rows_v, sem).wait()

    w_sq = [[wgt[4 * s + q] for q in range(4)] for s in range(SLOTS)]

    def blend_chunk(cc, carry):
        sl = pl.ds(cc * LANES, LANES)
        for s in range(SLOTS):
            res_v[s, sl] = (w_sq[s][0] * rows_v[4 * s + 0, sl]
                            + w_sq[s][1] * rows_v[4 * s + 1, sl]
                            + w_sq[s][2] * rows_v[4 * s + 2, sl]
                            + w_sq[s][3] * rows_v[4 * s + 3, sl])
        return carry

    lax.fori_loop(0, CCHUNKS, blend_chunk, 0)

    for s in range(SLOTS):
        @pl.when(SLOTS * wid + s < NPIX)
        def _(s=s):
            pltpu.async_copy(res_v.at[pl.ds(s, 1)],
                             out_hbm.at[pl.ds(SLOTS * wid + s, 1)], sem)

    for s in range(SLOTS):
        @pl.when(SLOTS * wid + s < NPIX)
        def _(s=s):
            pltpu.make_async_copy(res_v.at[pl.ds(s, 1)],
                                  out_hbm.at[pl.ds(SLOTS * wid + s, 1)],
                                  sem).wait()


def kernel(img, rois):
    pooled = _roi_pool_sc(img.reshape(H * W, C), rois.reshape(-1))
    return pooled.reshape(1, PH, PW, C)


# speedup vs baseline: 1.0037x; 1.0037x over previous
import functools

import jax
import jax.numpy as jnp
from jax import lax
from jax.experimental import pallas as pl
from jax.experimental.pallas import tpu as pltpu
from jax.experimental.pallas import tpu_sc as plsc

H, W, C = 50, 50, 512
PH, PW = 7, 7
NPIX = PH * PW
NWORK = 16
SLOTS = 4
LANES = 16
CCHUNKS = C // LANES


@functools.partial(
    pl.kernel,
    out_type=jax.ShapeDtypeStruct((PH, PW, C), jnp.float32),
    mesh=plsc.VectorSubcoreMesh(core_axis_name="c", subcore_axis_name="s",
                                num_cores=1),
    scratch_types=[
        pltpu.VMEM((LANES,), jnp.float32),
        pltpu.VMEM((LANES,), jnp.int32),
        pltpu.VMEM((LANES, C), jnp.float32),
        pltpu.VMEM((SLOTS, C), jnp.float32),
        pltpu.SemaphoreType.DMA,
    ],
)
def _roi_pool_sc(img_hbm, rois_hbm, out_hbm, roi_v, idx_v, rows_v, res_v,
                 sem):
    wid = lax.axis_index("s")
    lanes = lax.iota(jnp.int32, LANES)

    pltpu.sync_copy(rois_hbm.at[pl.ds(0, LANES)], roi_v)
    rvi = roi_v[...].astype(jnp.int32)
    x0c = rvi[0]
    y0c = rvi[1]
    in_w = rvi[2] - x0c + 1
    in_h = rvi[3] - y0c + 1
    d_w = jnp.full((LANES,), in_w.astype(jnp.float32)) / jnp.float32(PW)
    d_h = jnp.full((LANES,), in_h.astype(jnp.float32)) / jnp.float32(PH)

    s_l = lax.div(lanes, 4)
    q_l = lax.rem(lanes, 4)
    p_l = jnp.minimum(SLOTS * wid + s_l, NPIX - 1)
    i_l = lax.div(p_l, PW)
    j_l = lax.rem(p_l, PW)
    sy = i_l.astype(jnp.float32) * d_h
    sx = j_l.astype(jnp.float32) * d_w
    fy0 = sy.astype(jnp.int32)
    fx0 = sx.astype(jnp.int32)
    wy = sy - fy0.astype(jnp.float32)
    wx = sx - fx0.astype(jnp.float32)
    fy1 = jnp.minimum(fy0 + 1, in_h - 1)
    fx1 = jnp.minimum(fx0 + 1, in_w - 1)
    yy = y0c + jnp.where(q_l < 2, fy0, fy1)
    xx = x0c + jnp.where(lax.rem(q_l, 2) == 0, fx0, fx1)
    idx_v[...] = yy * W + xx

    wgt = jnp.where(q_l < 2, 1.0 - wy, wy) * jnp.where(
        lax.rem(q_l, 2) == 0, 1.0 - wx, wx)

    pltpu.async_copy(img_hbm.at[idx_v], rows_v, sem).wait()

    w_sq = [[wgt[4 * s + q] for q in range(4)] for s in range(SLOTS)]

    def blend_chunk(cc, carry):
        sl = pl.ds(cc * LANES, LANES)
        for s in range(SLOTS):
            res_v[s, sl] = (w_sq[s][0] * rows_v[4 * s + 0, sl]
                            + w_sq[s][1] * rows_v[4 * s + 1, sl]
                            + w_sq[s][2] * rows_v[4 * s + 2, sl]
                            + w_sq[s][3] * rows_v[4 * s + 3, sl])
        return carry

    lax.fori_loop(0, CCHUNKS, blend_chunk, 0)

    for s in range(SLOTS):
        @pl.when(SLOTS * wid + s < NPIX)
        def _(s=s):
            p = SLOTS * wid + s
            pltpu.async_copy(res_v.at[s], out_hbm.at[lax.div(p, PW),
                                                     lax.rem(p, PW)], sem)

    for s in range(SLOTS):
        @pl.when(SLOTS * wid + s < NPIX)
        def _(s=s):
            p = SLOTS * wid + s
            pltpu.make_async_copy(res_v.at[s],
                                  out_hbm.at[lax.div(p, PW), lax.rem(p, PW)],
                                  sem).wait()


def kernel(img, rois):
    rois16 = lax.slice(rois, (0, 0), (4, 4)).reshape(LANES)
    pooled = _roi_pool_sc(img.reshape(H * W, C), rois16)
    return pooled.reshape(1, PH, PW, C)


# speedup vs baseline: 1.4518x; 1.4465x over previous
import functools

import jax
import jax.numpy as jnp
from jax import lax
from jax.experimental import pallas as pl
from jax.experimental.pallas import tpu as pltpu
from jax.experimental.pallas import tpu_sc as plsc

H, W, C = 50, 50, 512
PH, PW = 7, 7
NPIX = PH * PW
NWORK = 16
SLOTS = 4
LANES = 16
CCHUNKS = C // LANES


@functools.partial(
    pl.kernel,
    out_type=jax.ShapeDtypeStruct((1, PH, PW, C), jnp.float32),
    mesh=plsc.VectorSubcoreMesh(core_axis_name="c", subcore_axis_name="s",
                                num_cores=1),
    scratch_types=[
        pltpu.VMEM((LANES,), jnp.float32),
        pltpu.VMEM((LANES,), jnp.int32),
        pltpu.VMEM((LANES,), jnp.int32),
        pltpu.VMEM((LANES, C), jnp.float32),
        pltpu.VMEM((SLOTS, C), jnp.float32),
        pltpu.SemaphoreType.DMA,
    ],
)
def _roi_pool_sc(img_hbm, rois_hbm, out_hbm, roi_v, iy_v, ix_v, rows_v,
                 res_v, sem):
    wid = lax.axis_index("s")
    lanes = lax.iota(jnp.int32, LANES)

    pltpu.sync_copy(rois_hbm.at[pl.ds(0, LANES)], roi_v)
    rvi = roi_v[...].astype(jnp.int32)
    x0c = rvi[0]
    y0c = rvi[1]
    in_w = rvi[2] - x0c + 1
    in_h = rvi[3] - y0c + 1
    d_w = jnp.full((LANES,), in_w.astype(jnp.float32)) / jnp.float32(PW)
    d_h = jnp.full((LANES,), in_h.astype(jnp.float32)) / jnp.float32(PH)

    s_l = lax.div(lanes, 4)
    q_l = lax.rem(lanes, 4)
    p_l = jnp.minimum(SLOTS * wid + s_l, NPIX - 1)
    i_l = lax.div(p_l, PW)
    j_l = lax.rem(p_l, PW)
    sy = i_l.astype(jnp.float32) * d_h
    sx = j_l.astype(jnp.float32) * d_w
    fy0 = sy.astype(jnp.int32)
    fx0 = sx.astype(jnp.int32)
    wy = sy - fy0.astype(jnp.float32)
    wx = sx - fx0.astype(jnp.float32)
    fy1 = jnp.minimum(fy0 + 1, in_h - 1)
    fx1 = jnp.minimum(fx0 + 1, in_w - 1)
    iy_v[...] = y0c + jnp.where(q_l < 2, fy0, fy1)
    ix_v[...] = x0c + jnp.where(lax.rem(q_l, 2) == 0, fx0, fx1)

    wgt = jnp.where(q_l < 2, 1.0 - wy, wy) * jnp.where(
        lax.rem(q_l, 2) == 0, 1.0 - wx, wx)

    iy = iy_v[...]
    ix = ix_v[...]
    for k in range(LANES):
        pltpu.async_copy(img_hbm.at[0, iy[k], ix[k]], rows_v.at[k], sem)
    for k in range(LANES):
        pltpu.make_async_copy(img_hbm.at[0, iy[k], ix[k]], rows_v.at[k],
                              sem).wait()

    w_sq = [[wgt[4 * s + q] for q in range(4)] for s in range(SLOTS)]

    def blend_chunk(cc, carry):
        sl = pl.ds(cc * LANES, LANES)
        for s in range(SLOTS):
            res_v[s, sl] = (w_sq[s][0] * rows_v[4 * s + 0, sl]
                            + w_sq[s][1] * rows_v[4 * s + 1, sl]
                            + w_sq[s][2] * rows_v[4 * s + 2, sl]
                            + w_sq[s][3] * rows_v[4 * s + 3, sl])
        return carry

    lax.fori_loop(0, CCHUNKS, blend_chunk, 0)

    for s in range(SLOTS):
        @pl.when(SLOTS * wid + s < NPIX)
        def _(s=s):
            p = SLOTS * wid + s
            pltpu.async_copy(res_v.at[s],
                             out_hbm.at[0, lax.div(p, PW), lax.rem(p, PW)],
                             sem)

    for s in range(SLOTS):
        @pl.when(SLOTS * wid + s < NPIX)
        def _(s=s):
            p = SLOTS * wid + s
            pltpu.make_async_copy(res_v.at[s],
                                  out_hbm.at[0, lax.div(p, PW),
                                             lax.rem(p, PW)],
                                  sem).wait()


def kernel(img, rois):
    rois16 = lax.slice(rois, (0, 0), (4, 4)).reshape(LANES)
    return _roi_pool_sc(img, rois16)
